# baseline (speedup 1.0000x reference)
import functools

import jax
import jax.numpy as jnp
from jax.experimental import pallas as pl
from jax.experimental.pallas import tpu as pltpu


def _copy_body(w_hbm, o_hbm, out_sem, *, B):
    copies = [
        pltpu.make_async_copy(w_hbm, o_hbm.at[b], out_sem.at[b])
        for b in range(B)
    ]
    for c in copies:
        c.start()
    for c in copies:
        c.wait()


def kernel(x, W):
    B = x.shape[0]
    T, H = W.shape
    body = functools.partial(_copy_body, B=B)
    return pl.pallas_call(
        body,
        in_specs=[pl.BlockSpec(memory_space=pl.ANY)],
        out_specs=pl.BlockSpec(memory_space=pl.ANY),
        out_shape=jax.ShapeDtypeStruct((B, T, H), W.dtype),
        scratch_shapes=[
            pltpu.SemaphoreType.DMA((B,)),
        ],
    )(W)


# speedup vs baseline: 74.0363x; 74.0363x over previous
import functools

import jax
import jax.numpy as jnp
from jax.experimental import pallas as pl
from jax.experimental.pallas import tpu as pltpu


def _copy_body(w_hbm, o_hbm, w_vmem, in_sem, out_sem, *, B, KI, KO, CTI, CTO):
    ins = [
        pltpu.make_async_copy(
            w_hbm.at[pl.ds(k * CTI, CTI), :],
            w_vmem.at[pl.ds(k * CTI, CTI), :],
            in_sem.at[k],
        )
        for k in range(KI)
    ]
    for c in ins:
        c.start()
    r = KI // KO
    outs = []
    for k in range(KO):
        for j in range(r):
            ins[k * r + j].wait()
        for b in range(B):
            c = pltpu.make_async_copy(
                w_vmem.at[pl.ds(k * CTO, CTO), :],
                o_hbm.at[b, pl.ds(k * CTO, CTO), :],
                out_sem.at[b],
            )
            c.start()
            outs.append(c)
    for c in outs:
        c.wait()


def kernel(x, W):
    B = x.shape[0]
    T, H = W.shape
    KI = 8
    KO = 4
    CTI = T // KI
    CTO = T // KO
    body = functools.partial(_copy_body, B=B, KI=KI, KO=KO, CTI=CTI, CTO=CTO)
    return pl.pallas_call(
        body,
        in_specs=[pl.BlockSpec(memory_space=pl.ANY)],
        out_specs=pl.BlockSpec(memory_space=pl.ANY),
        out_shape=jax.ShapeDtypeStruct((B, T, H), W.dtype),
        scratch_shapes=[
            pltpu.VMEM((T, H), W.dtype),
            pltpu.SemaphoreType.DMA((KI,)),
            pltpu.SemaphoreType.DMA((B,)),
        ],
    )(W)
